# baseline (speedup 1.0000x reference)
import functools

import jax
import jax.numpy as jnp
from jax import lax
from jax.experimental import pallas as pl
from jax.experimental.pallas import tpu as pltpu
from jax.experimental.pallas import tpu_sc as plsc

VOCAB = 8192
D = 8192
K = 4
NBUF = 2


def _body(x_hbm, table_hbm, out_hbm, idx_v, buf0, buf1,
          gsem0, gsem1, ssem0, ssem1, *, nw, nchunks):
    bufs = (buf0, buf1)
    gsems = (gsem0, gsem1)
    ssems = (ssem0, ssem1)

    cid = lax.axis_index("c")
    sid = lax.axis_index("s")
    wid = sid * 2 + cid
    base = wid * nchunks

    pltpu.sync_copy(x_hbm.at[pl.ds(base, nchunks)], idx_v)

    def gather(j, b):
        return pltpu.make_async_copy(
            table_hbm.at[idx_v.at[j]], bufs[b], gsems[b])

    def scatter(j, b):
        return pltpu.make_async_copy(
            bufs[b], out_hbm.at[base + j], ssems[b])

    for b in range(NBUF):
        gather(b, b).start()
    for b in range(NBUF):
        gather(b, b).wait()
        scatter(b, b).start()

    def step(jj, carry):
        for b in range(NBUF):
            j = jj * NBUF + b
            scatter(j, b).wait()
            gather(j, b).start()
            gather(j, b).wait()
            scatter(j, b).start()
        return carry

    lax.fori_loop(1, nchunks // NBUF, step, 0)

    for b in range(NBUF):
        scatter(0, b).wait()


def kernel(x, table):
    B, S = x.shape
    n = B * S
    info = plsc.get_sparse_core_info()
    nw = info.num_cores * info.num_subcores
    nchunks = n // (nw * K)

    xf = x.reshape(nw * nchunks, K).astype(jnp.int32)

    mesh = plsc.VectorSubcoreMesh(core_axis_name="c", subcore_axis_name="s")
    out = pl.kernel(
        functools.partial(_body, nw=nw, nchunks=nchunks),
        out_type=jax.ShapeDtypeStruct((nw * nchunks, K, D), jnp.float32),
        mesh=mesh,
        scratch_types=[
            pltpu.VMEM((nchunks, K), jnp.int32),
            pltpu.VMEM((K, D), jnp.float32),
            pltpu.VMEM((K, D), jnp.float32),
            pltpu.SemaphoreType.DMA,
            pltpu.SemaphoreType.DMA,
            pltpu.SemaphoreType.DMA,
            pltpu.SemaphoreType.DMA,
        ],
    )(xf, table)
    return out.reshape(B, S, VOCAB)


# speedup vs baseline: 2.4443x; 2.4443x over previous
import functools

import jax
import jax.numpy as jnp
from jax import lax
from jax.experimental import pallas as pl
from jax.experimental.pallas import tpu as pltpu
from jax.experimental.pallas import tpu_sc as plsc

VOCAB = 8192
D = 8192
K = 4
NBUF = 2


def _body(x_hbm, table_hbm, out_hbm, idx_v, buf0, buf1,
          gsem0, gsem1, ssem0, ssem1, *, nw, nchunks, seq):
    bufs = (buf0, buf1)
    gsems = (gsem0, gsem1)
    ssems = (ssem0, ssem1)

    cid = lax.axis_index("c")
    sid = lax.axis_index("s")
    wid = sid * 2 + cid
    base = wid * nchunks
    wpb = seq // (nchunks * K)
    b_out = wid // wpb
    s_base = (wid % wpb) * (nchunks * K)

    pltpu.sync_copy(x_hbm.at[pl.ds(base, nchunks)], idx_v)

    def gather(j, b):
        return pltpu.make_async_copy(
            table_hbm.at[idx_v.at[j]], bufs[b], gsems[b])

    def scatter(j, b):
        return pltpu.make_async_copy(
            bufs[b], out_hbm.at[b_out, pl.ds(s_base + j * K, K)], ssems[b])

    for b in range(NBUF):
        gather(b, b).start()
    for b in range(NBUF):
        gather(b, b).wait()
        scatter(b, b).start()

    def step(jj, carry):
        for b in range(NBUF):
            j = jj * NBUF + b
            scatter(j, b).wait()
            gather(j, b).start()
            gather(j, b).wait()
            scatter(j, b).start()
        return carry

    lax.fori_loop(1, nchunks // NBUF, step, 0)

    for b in range(NBUF):
        scatter(0, b).wait()


def kernel(x, table):
    B, S = x.shape
    n = B * S
    info = plsc.get_sparse_core_info()
    nw = info.num_cores * info.num_subcores
    nchunks = n // (nw * K)

    xf = x.reshape(nw * nchunks, K).astype(jnp.int32)

    mesh = plsc.VectorSubcoreMesh(core_axis_name="c", subcore_axis_name="s")
    out = pl.kernel(
        functools.partial(_body, nw=nw, nchunks=nchunks, seq=S),
        out_type=jax.ShapeDtypeStruct((B, S, D), jnp.float32),
        mesh=mesh,
        scratch_types=[
            pltpu.VMEM((nchunks, K), jnp.int32),
            pltpu.VMEM((K, D), jnp.float32),
            pltpu.VMEM((K, D), jnp.float32),
            pltpu.SemaphoreType.DMA,
            pltpu.SemaphoreType.DMA,
            pltpu.SemaphoreType.DMA,
            pltpu.SemaphoreType.DMA,
        ],
    )(xf, table)
    return out


# speedup vs baseline: 2.4542x; 1.0040x over previous
import functools

import jax
import jax.numpy as jnp
from jax import lax
from jax.experimental import pallas as pl
from jax.experimental.pallas import tpu as pltpu
from jax.experimental.pallas import tpu_sc as plsc

VOCAB = 8192
D = 8192
K = 4
NBUF = 3


def _body(x_hbm, table_hbm, out_hbm, idx_v, *rest, nw, nchunks, seq):
    bufs = rest[:NBUF]
    gsems = rest[NBUF:2 * NBUF]
    ssems = rest[2 * NBUF:3 * NBUF]

    cid = lax.axis_index("c")
    sid = lax.axis_index("s")
    wid = sid * 2 + cid
    base = wid * nchunks
    wpb = seq // (nchunks * K)
    b_out = wid // wpb
    s_base = (wid % wpb) * (nchunks * K)

    pltpu.sync_copy(x_hbm.at[pl.ds(base, nchunks)], idx_v)

    def gather(j, b):
        return pltpu.make_async_copy(
            table_hbm.at[idx_v.at[j]], bufs[b], gsems[b])

    def scatter(j, b):
        return pltpu.make_async_copy(
            bufs[b], out_hbm.at[b_out, pl.ds(s_base + j * K, K)], ssems[b])

    for b in range(NBUF):
        gather(b, b).start()
    for b in range(NBUF):
        gather(b, b).wait()
        scatter(b, b).start()

    def step(jj, carry):
        for b in range(NBUF):
            j = jj * NBUF + b
            scatter(j, b).wait()
            gather(j, b).start()
            gather(j, b).wait()
            scatter(j, b).start()
        return carry

    nfull = (nchunks // NBUF) * NBUF
    lax.fori_loop(1, nfull // NBUF, step, 0)

    for j in range(nfull, nchunks):
        b = j % NBUF
        scatter(j, b).wait()
        gather(j, b).start()
        gather(j, b).wait()
        scatter(j, b).start()

    for b in range(NBUF):
        scatter(0, b).wait()


def kernel(x, table):
    B, S = x.shape
    n = B * S
    info = plsc.get_sparse_core_info()
    nw = info.num_cores * info.num_subcores
    nchunks = n // (nw * K)

    xf = x.reshape(nw * nchunks, K).astype(jnp.int32)

    mesh = plsc.VectorSubcoreMesh(core_axis_name="c", subcore_axis_name="s")
    out = pl.kernel(
        functools.partial(_body, nw=nw, nchunks=nchunks, seq=S),
        out_type=jax.ShapeDtypeStruct((B, S, D), jnp.float32),
        mesh=mesh,
        scratch_types=(
            [pltpu.VMEM((nchunks, K), jnp.int32)]
            + [pltpu.VMEM((K, D), jnp.float32)] * NBUF
            + [pltpu.SemaphoreType.DMA] * (2 * NBUF)
        ),
    )(xf, table)
    return out


# speedup vs baseline: 2.4585x; 1.0018x over previous
import functools

import jax
import jax.numpy as jnp
from jax import lax
from jax.experimental import pallas as pl
from jax.experimental.pallas import tpu as pltpu
from jax.experimental.pallas import tpu_sc as plsc

VOCAB = 8192
D = 8192
K = 4
NBUF = 3


def _body(x_hbm, table_hbm, out_hbm, idx_v, *rest, nw, nchunks, seq):
    bufs = rest[:NBUF]
    gsems = rest[NBUF:2 * NBUF]
    ssems = rest[2 * NBUF:3 * NBUF]

    cid = lax.axis_index("c")
    sid = lax.axis_index("s")
    wid = sid * 2 + cid
    base = wid * nchunks
    wpb = seq // (nchunks * K)
    b_out = wid // wpb
    s_base = (wid % wpb) * (nchunks * K)

    pltpu.sync_copy(x_hbm.at[pl.ds(base, nchunks)], idx_v)

    def gather(j, b):
        return pltpu.make_async_copy(
            table_hbm.at[idx_v.at[j]], bufs[b], gsems[b])

    def scatter(j, b):
        return pltpu.make_async_copy(
            bufs[b], out_hbm.at[b_out, pl.ds(s_base + j * K, K)], ssems[b])

    gather(0, 0).start()
    gather(1, 1).start()

    def chunk_step(j, b):
        bn = (b + 2) % NBUF
        gather(j, b).wait()
        scatter(j, b).start()
        scatter(j - 1, bn).wait()
        gather(j + 2, bn).start()

    gather(0, 0).wait()
    scatter(0, 0).start()
    gather(2, 2).start()
    chunk_step(1, 1)
    chunk_step(2, 2)

    def step(jj, carry):
        for b in range(NBUF):
            chunk_step(jj * NBUF + b, b)
        return carry

    lax.fori_loop(1, (nchunks - 2) // NBUF, step, 0)

    for j in range(nchunks - 2, nchunks):
        b = j % NBUF
        gather(j, b).wait()
        scatter(j, b).start()

    for b in range(NBUF):
        scatter(0, b).wait()


def kernel(x, table):
    B, S = x.shape
    n = B * S
    info = plsc.get_sparse_core_info()
    nw = info.num_cores * info.num_subcores
    nchunks = n // (nw * K)

    xf = x.reshape(nw * nchunks, K).astype(jnp.int32)

    mesh = plsc.VectorSubcoreMesh(core_axis_name="c", subcore_axis_name="s")
    out = pl.kernel(
        functools.partial(_body, nw=nw, nchunks=nchunks, seq=S),
        out_type=jax.ShapeDtypeStruct((B, S, D), jnp.float32),
        mesh=mesh,
        scratch_types=(
            [pltpu.VMEM((nchunks, K), jnp.int32)]
            + [pltpu.VMEM((K, D), jnp.float32)] * NBUF
            + [pltpu.SemaphoreType.DMA] * (2 * NBUF)
        ),
    )(xf, table)
    return out
